# baseline (speedup 1.0000x reference)
import jax
import jax.numpy as jnp
from jax import lax
from jax.experimental import pallas as pl
from jax.experimental.pallas import tpu as pltpu
from jax.experimental.pallas import tpu_sc as plsc

N = 50000
E = 1600000
NP = 51200
NC = 2
NS = 16
NW = NC * NS
GRP = 512
CG = 2
CHUNKS = 98
EP = NS * CHUNKS * CG * GRP
ROWS_PER_TILE = NP // NS
HF = 16


def _sc_mesh():
    return plsc.VectorSubcoreMesh(core_axis_name="c", subcore_axis_name="s")


CE = CG * GRP


def _scale_rows(rows_v, ew_v, r, d):
    rbase = r * CE
    ebase = d * CE

    @pl.loop(0, CE // 16, unroll=4)
    def _scale(g):
        ew16 = ew_v[pl.ds(ebase + g * 16, 16)]
        for k in range(16):
            e = rbase + g * 16 + k
            w16 = ew16.at[jnp.full((16,), k, jnp.int32)].get(
                mode="promise_in_bounds")
            v0 = rows_v[e, pl.ds(0, HF)]
            rows_v[e, pl.ds(0, HF)] = v0 * w16


def _msg_body(y_hbm, row_hbm, col_hbm, ew_hbm, zeros_hbm, zp_hbm,
              row_v, col_v, ew_v, rows_v, z_sh, isem, gsem, ssem):
    c = lax.axis_index("c")
    s = lax.axis_index("s")
    base_row = s * ROWS_PER_TILE
    pltpu.sync_copy(zeros_hbm, z_sh.at[pl.ds(base_row, ROWS_PER_TILE)])
    plsc.subcore_barrier()

    def start_idx(ch, d):
        pltpu.async_copy(row_hbm.at[c, s, ch], row_v.at[d], isem)
        pltpu.async_copy(col_hbm.at[s, ch], col_v.at[d], isem)
        pltpu.async_copy(ew_hbm.at[s, pl.ds(ch * CE, CE)],
                         ew_v.at[pl.ds(d * CE, CE)], isem)

    def wait_idx(d):
        pltpu.make_async_copy(row_hbm.at[c, s, 0], row_v.at[d], isem).wait()
        pltpu.make_async_copy(col_hbm.at[s, 0], col_v.at[d], isem).wait()
        pltpu.make_async_copy(ew_hbm.at[s, pl.ds(0, CE)],
                              ew_v.at[pl.ds(d * CE, CE)], isem).wait()

    def start_gather(r, d):
        for j in range(CG):
            pltpu.async_copy(y_hbm.at[row_v.at[d, j]],
                             rows_v.at[pl.ds(r * CE + j * GRP, GRP)], gsem)

    def wait_gather(r):
        pltpu.make_async_copy(y_hbm.at[pl.ds(0, CE)],
                              rows_v.at[pl.ds(r * CE, CE)], gsem).wait()

    def start_scatter(r, d):
        for j in range(CG):
            pltpu.async_copy(rows_v.at[pl.ds(r * CE + j * GRP, GRP)],
                             z_sh.at[col_v.at[d, j]], ssem, add=True)

    def wait_scatter(r):
        pltpu.make_async_copy(rows_v.at[pl.ds(r * CE, CE)],
                              z_sh.at[pl.ds(0, CE)], ssem).wait()

    start_idx(0, 0)
    start_idx(1, 1)
    wait_idx(0)
    start_gather(0, 0)

    @pl.loop(0, CHUNKS)
    def _chunk(i):
        r = lax.rem(i, 3)
        rn = lax.rem(i + 1, 3)
        d = lax.bitwise_and(i, 3)
        dn = lax.bitwise_and(i + 1, 3)
        d2 = lax.bitwise_and(i + 2, 3)
        wait_gather(r)

        @pl.when(i >= 2)
        def _():
            wait_scatter(lax.rem(i + 1, 3))

        @pl.when(i + 1 < CHUNKS)
        def _():
            wait_idx(dn)
            start_gather(rn, dn)

        @pl.when(i + 2 < CHUNKS)
        def _():
            start_idx(i + 2, d2)

        _scale_rows(rows_v, ew_v, r, d)
        start_scatter(r, d)

    wait_scatter(lax.rem(CHUNKS - 2, 3))
    wait_scatter(lax.rem(CHUNKS - 1, 3))

    plsc.subcore_barrier()
    pltpu.sync_copy(z_sh.at[pl.ds(base_row, ROWS_PER_TILE)],
                    zp_hbm.at[c, pl.ds(base_row, ROWS_PER_TILE)])


def _msg_call(y2, row_r2, col_r2, ew_r2, zeros_rt):
    k = pl.kernel(
        _msg_body,
        out_type=jax.ShapeDtypeStruct((NC, NP, HF), jnp.float32),
        mesh=_sc_mesh(),
        compiler_params=pltpu.CompilerParams(use_tc_tiling_on_sc=False),
        scratch_types=[
            pltpu.VMEM((4, CG, GRP), jnp.int32),
            pltpu.VMEM((4, CG, GRP), jnp.int32),
            pltpu.VMEM((4 * CE,), jnp.float32),
            pltpu.VMEM((3 * CE, HF), jnp.float32),
            pltpu.VMEM_SHARED((NP, HF), jnp.float32),
            pltpu.SemaphoreType.DMA,
            pltpu.SemaphoreType.DMA,
            pltpu.SemaphoreType.DMA,
        ],
    )
    args = [pltpu.with_memory_space_constraint(a, pltpu.HBM)
            for a in (y2, row_r2, col_r2, ew_r2, zeros_rt)]
    return k(*args)


DCHUNKS = CHUNKS // 2


def _deg_body(col_hbm, ew_hbm, zeros_hbm, zp_hbm,
              col_v, ew_v, rows_v, z_sh, isem, ssem):
    c = lax.axis_index("c")
    s = lax.axis_index("s")
    base_row = s * ROWS_PER_TILE
    pltpu.sync_copy(zeros_hbm, z_sh.at[pl.ds(base_row, ROWS_PER_TILE)])
    plsc.subcore_barrier()

    def start_idx(ch, d):
        chg = c * DCHUNKS + ch
        pltpu.async_copy(col_hbm.at[s, chg], col_v.at[d], isem)
        pltpu.async_copy(ew_hbm.at[s, pl.ds(chg * CE, CE)],
                         ew_v.at[pl.ds(d * CE, CE)], isem)

    def wait_idx(d):
        pltpu.make_async_copy(col_hbm.at[s, 0], col_v.at[d], isem).wait()
        pltpu.make_async_copy(ew_hbm.at[s, pl.ds(0, CE)],
                              ew_v.at[pl.ds(d * CE, CE)], isem).wait()

    def start_scatter(r, d):
        for j in range(CG):
            pltpu.async_copy(rows_v.at[pl.ds(r * CE + j * GRP, GRP)],
                             z_sh.at[col_v.at[d, j]], ssem, add=True)

    def wait_scatter(r):
        pltpu.make_async_copy(rows_v.at[pl.ds(r * CE, CE)],
                              z_sh.at[pl.ds(0, CE)], ssem).wait()

    start_idx(0, 0)
    start_idx(1, 1)

    @pl.loop(0, DCHUNKS)
    def _chunk(i):
        r = lax.bitwise_and(i, 1)
        d = lax.bitwise_and(i, 3)

        @pl.when(i >= 2)
        def _():
            wait_scatter(lax.bitwise_and(i, 1))

        wait_idx(d)

        rbase = r * CE
        ebase = d * CE

        @pl.loop(0, CE // 16)
        def _fill(g):
            ew16 = ew_v[pl.ds(ebase + g * 16, 16)]
            for k in range(16):
                e = rbase + g * 16 + k
                w16 = ew16.at[jnp.full((16,), k, jnp.int32)].get(
                    mode="promise_in_bounds")
                v0 = rows_v[e, pl.ds(0, HF)]
                rows_v[e, pl.ds(0, HF)] = v0 * 0.0 + w16

        @pl.when(i + 2 < DCHUNKS)
        def _():
            start_idx(i + 2, lax.bitwise_and(i + 2, 3))

        start_scatter(r, d)

    wait_scatter(lax.bitwise_and(DCHUNKS - 2, 1))
    wait_scatter(lax.bitwise_and(DCHUNKS - 1, 1))

    plsc.subcore_barrier()
    pltpu.sync_copy(z_sh.at[pl.ds(base_row, ROWS_PER_TILE)],
                    zp_hbm.at[c, pl.ds(base_row, ROWS_PER_TILE)])


def _deg_call(col_r2, ew_r2, zeros_rt):
    k = pl.kernel(
        _deg_body,
        out_type=jax.ShapeDtypeStruct((NC, NP, HF), jnp.float32),
        mesh=_sc_mesh(),
        compiler_params=pltpu.CompilerParams(use_tc_tiling_on_sc=False),
        scratch_types=[
            pltpu.VMEM((4, CG, GRP), jnp.int32),
            pltpu.VMEM((4 * CE,), jnp.float32),
            pltpu.VMEM((2 * CE, HF), jnp.float32),
            pltpu.VMEM_SHARED((NP, HF), jnp.float32),
            pltpu.SemaphoreType.DMA,
            pltpu.SemaphoreType.DMA,
        ],
    )
    args = [pltpu.with_memory_space_constraint(a, pltpu.HBM)
            for a in (col_r2, ew_r2, zeros_rt)]
    return k(*args)


BLK = 3200
GRID = NP // BLK


def _mm(a, b):
    return lax.dot_general(a, b, (((1,), (0,)), ((), ())),
                           preferred_element_type=jnp.float32)


def _tc_ab_body(c_ref, x_ref, degp_ref, we1_ref, be1_ref, we2_ref, be2_ref,
                w1a_ref, w1b_ref, y1s_ref, dinv_ref, xw1_ref):
    emb = _mm(jax.nn.relu(_mm(c_ref[...], we1_ref[...]) + be1_ref[...]),
              we2_ref[...]) + be2_ref[...]
    xw1 = _mm(x_ref[...], w1a_ref[...]) + _mm(emb, w1b_ref[...])
    xw1_ref[...] = xw1
    deg = degp_ref[0, :, 0:1] + degp_ref[1, :, 0:1] + 1.0
    dinv = lax.rsqrt(deg)
    dinv_ref[...] = dinv
    y1 = xw1 * dinv
    y1s_ref[0] = y1[:, :HF]
    y1s_ref[1] = y1[:, HF:]


def _tc_c_body(zp_ref, dinv_ref, xw1_ref, b1_ref, w2_ref, y2s_ref, xw2_ref):
    dinv = dinv_ref[...]
    z1 = jnp.concatenate([zp_ref[0], zp_ref[1]], axis=1)
    h1 = jax.nn.relu((z1 + dinv * xw1_ref[...]) * dinv + b1_ref[...])
    xw2 = _mm(h1, w2_ref[...])
    xw2_ref[...] = xw2
    y2 = xw2 * dinv
    y2s_ref[0] = y2[:, :HF]
    y2s_ref[1] = y2[:, HF:]


def _tc_d_body(zp_ref, dinv_ref, xw2_ref, b2_ref, wf_ref, bf_ref, out_ref):
    dinv = dinv_ref[...]
    z2 = jnp.concatenate([zp_ref[0], zp_ref[1]], axis=1)
    h2 = jax.nn.relu((z2 + dinv * xw2_ref[...]) * dinv + b2_ref[...])
    out_ref[...] = _mm(h2, wf_ref[...]) + bf_ref[...]


def _row_spec(width):
    return pl.BlockSpec((BLK, width), lambda i: (i, 0))


def _part_spec(width):
    return pl.BlockSpec((NC, BLK, width), lambda i: (0, i, 0))


def _full_spec(shape):
    nd = len(shape)
    return pl.BlockSpec(shape, lambda i: (0,) * nd)


def kernel(x, c, ei, ew, W_enc1, b_enc1, W_enc2, b_enc2, W1, b1, W2, b2, Wf, bf):
    f32 = jnp.float32
    x = x.astype(f32)
    c = c.astype(f32)
    ew = ew.astype(f32)

    pad_e = EP - E
    row = jnp.concatenate([ei[0].astype(jnp.int32),
                           jnp.zeros((pad_e,), jnp.int32)])
    col = jnp.concatenate([ei[1].astype(jnp.int32),
                           jnp.zeros((pad_e,), jnp.int32)])
    ewp = jnp.concatenate([ew, jnp.zeros((pad_e,), f32)])
    row_t = row.reshape(NS, CHUNKS, CG, GRP)
    row_r2 = jnp.stack([row_t, row_t + NP])
    col_r2 = col.reshape(NS, CHUNKS, CG, GRP)
    ew_r2 = ewp.reshape(NS, CHUNKS * CE)

    pad_n = NP - N
    xp = jnp.concatenate([x, jnp.zeros((pad_n, x.shape[1]), f32)])
    cp = jnp.concatenate([c, jnp.zeros((pad_n, c.shape[1]), f32)])

    zeros_rt = jnp.zeros((ROWS_PER_TILE, HF), f32)

    be1 = b_enc1.reshape(1, -1)
    be2 = b_enc2.reshape(1, -1)
    W1a = W1[:3]
    W1b = W1[3:]
    b1r = b1.reshape(1, -1)
    b2r = b2.reshape(1, -1)
    bfr = bf.reshape(1, -1)

    degp = _deg_call(col_r2, ew_r2, zeros_rt)

    y1s, dinv, xw1 = pl.pallas_call(
        _tc_ab_body,
        grid=(GRID,),
        in_specs=[_row_spec(2), _row_spec(3), _part_spec(HF),
                  _full_spec(W_enc1.shape), _full_spec(be1.shape),
                  _full_spec(W_enc2.shape), _full_spec(be2.shape),
                  _full_spec(W1a.shape), _full_spec(W1b.shape)],
        out_specs=[_part_spec(HF), _row_spec(1), _row_spec(32)],
        out_shape=[jax.ShapeDtypeStruct((NC, NP, HF), f32),
                   jax.ShapeDtypeStruct((NP, 1), f32),
                   jax.ShapeDtypeStruct((NP, 32), f32)],
    )(cp, xp, degp, W_enc1, be1, W_enc2, be2, W1a, W1b)

    z1p = _msg_call(y1s.reshape(2 * NP, HF), row_r2, col_r2, ew_r2, zeros_rt)

    y2s, xw2 = pl.pallas_call(
        _tc_c_body,
        grid=(GRID,),
        in_specs=[_part_spec(HF), _row_spec(1), _row_spec(32),
                  _full_spec(b1r.shape), _full_spec(W2.shape)],
        out_specs=[_part_spec(HF), _row_spec(32)],
        out_shape=[jax.ShapeDtypeStruct((NC, NP, HF), f32),
                   jax.ShapeDtypeStruct((NP, 32), f32)],
    )(z1p, dinv, xw1, b1r, W2)

    z2p = _msg_call(y2s.reshape(2 * NP, HF), row_r2, col_r2, ew_r2, zeros_rt)

    out = pl.pallas_call(
        _tc_d_body,
        grid=(GRID,),
        in_specs=[_part_spec(HF), _row_spec(1), _row_spec(32),
                  _full_spec(b2r.shape), _full_spec(Wf.shape),
                  _full_spec(bfr.shape)],
        out_specs=_row_spec(1),
        out_shape=jax.ShapeDtypeStruct((NP, 1), f32),
    )(z2p, dinv, xw2, b2r, Wf, bfr)

    return out[:N]


# speedup vs baseline: 1.4077x; 1.4077x over previous
import jax
import jax.numpy as jnp
from jax import lax
from jax.experimental import pallas as pl
from jax.experimental.pallas import tpu as pltpu
from jax.experimental.pallas import tpu_sc as plsc

N = 50000
E = 1600000
NP = 51200
NC = 2
NS = 16
NW = NC * NS
GRP = 512
CG = 2
CHUNKS = 98
EP = NS * CHUNKS * CG * GRP
ROWS_PER_TILE = NP // NS
HF = 16


def _sc_mesh():
    return plsc.VectorSubcoreMesh(core_axis_name="c", subcore_axis_name="s")


CE = CG * GRP


def _scale_rows(rows_v, ew_v, r, d):
    rbase = r * CE
    ebase = d * CE

    @pl.loop(0, CE // 16, unroll=2)
    def _scale(g):
        ew16 = ew_v[pl.ds(ebase + g * 16, 16)]
        for k in range(16):
            e = rbase + g * 16 + k
            w16 = ew16.at[jnp.full((16,), k, jnp.int32)].get(
                mode="promise_in_bounds")
            v0 = rows_v[e, pl.ds(0, HF)]
            rows_v[e, pl.ds(0, HF)] = v0 * w16


def _msg_body(y_hbm, row_hbm, col_hbm, ew_hbm, zeros_hbm, zp_hbm,
              row_v, col_v, ew_v, rows_v, z_sh, isem, gsem, ssem):
    c = lax.axis_index("c")
    s = lax.axis_index("s")
    base_row = s * ROWS_PER_TILE
    pltpu.sync_copy(zeros_hbm, z_sh.at[pl.ds(base_row, ROWS_PER_TILE)])
    plsc.subcore_barrier()

    def start_idx(ch, d):
        pltpu.async_copy(row_hbm.at[c, s, ch], row_v.at[d], isem)
        pltpu.async_copy(col_hbm.at[s, ch], col_v.at[d], isem)
        pltpu.async_copy(ew_hbm.at[s, pl.ds(ch * CE, CE)],
                         ew_v.at[pl.ds(d * CE, CE)], isem)

    def wait_idx(d):
        pltpu.make_async_copy(row_hbm.at[c, s, 0], row_v.at[d], isem).wait()
        pltpu.make_async_copy(col_hbm.at[s, 0], col_v.at[d], isem).wait()
        pltpu.make_async_copy(ew_hbm.at[s, pl.ds(0, CE)],
                              ew_v.at[pl.ds(d * CE, CE)], isem).wait()

    def start_gather(r, d):
        for j in range(CG):
            pltpu.async_copy(y_hbm.at[row_v.at[d, j]],
                             rows_v.at[pl.ds(r * CE + j * GRP, GRP)], gsem)

    def wait_gather(r):
        pltpu.make_async_copy(y_hbm.at[pl.ds(0, CE)],
                              rows_v.at[pl.ds(r * CE, CE)], gsem).wait()

    def start_scatter(r, d):
        for j in range(CG):
            pltpu.async_copy(rows_v.at[pl.ds(r * CE + j * GRP, GRP)],
                             z_sh.at[col_v.at[d, j]], ssem, add=True)

    def wait_scatter(r):
        pltpu.make_async_copy(rows_v.at[pl.ds(r * CE, CE)],
                              z_sh.at[pl.ds(0, CE)], ssem).wait()

    start_idx(0, 0)
    start_idx(1, 1)
    wait_idx(0)
    start_gather(0, 0)

    @pl.loop(0, CHUNKS)
    def _chunk(i):
        r = lax.rem(i, 3)
        rn = lax.rem(i + 1, 3)
        d = lax.bitwise_and(i, 3)
        dn = lax.bitwise_and(i + 1, 3)
        d2 = lax.bitwise_and(i + 2, 3)
        wait_gather(r)

        @pl.when(i >= 2)
        def _():
            wait_scatter(lax.rem(i + 1, 3))

        @pl.when(i + 1 < CHUNKS)
        def _():
            wait_idx(dn)
            start_gather(rn, dn)

        @pl.when(i + 2 < CHUNKS)
        def _():
            start_idx(i + 2, d2)

        _scale_rows(rows_v, ew_v, r, d)
        start_scatter(r, d)

    wait_scatter(lax.rem(CHUNKS - 2, 3))
    wait_scatter(lax.rem(CHUNKS - 1, 3))

    plsc.subcore_barrier()
    pltpu.sync_copy(z_sh.at[pl.ds(base_row, ROWS_PER_TILE)],
                    zp_hbm.at[c, pl.ds(base_row, ROWS_PER_TILE)])


def _msg_call(y2, row_r2, col_r2, ew_r2, zeros_rt):
    k = pl.kernel(
        _msg_body,
        out_type=jax.ShapeDtypeStruct((NC, NP, HF), jnp.float32),
        mesh=_sc_mesh(),
        compiler_params=pltpu.CompilerParams(use_tc_tiling_on_sc=False),
        scratch_types=[
            pltpu.VMEM((4, CG, GRP), jnp.int32),
            pltpu.VMEM((4, CG, GRP), jnp.int32),
            pltpu.VMEM((4 * CE,), jnp.float32),
            pltpu.VMEM((3 * CE, HF), jnp.float32),
            pltpu.VMEM_SHARED((NP, HF), jnp.float32),
            pltpu.SemaphoreType.DMA,
            pltpu.SemaphoreType.DMA,
            pltpu.SemaphoreType.DMA,
        ],
    )
    args = [pltpu.with_memory_space_constraint(a, pltpu.HBM)
            for a in (y2, row_r2, col_r2, ew_r2, zeros_rt)]
    return k(*args)


DCHUNKS = CHUNKS // 2


def _deg_body(col_hbm, ew_hbm, zeros_hbm, zp_hbm,
              col_v, ew_v, rows_v, z_sh, isem, ssem):
    c = lax.axis_index("c")
    s = lax.axis_index("s")
    base_row = s * ROWS_PER_TILE
    pltpu.sync_copy(zeros_hbm, z_sh.at[pl.ds(base_row, ROWS_PER_TILE)])
    plsc.subcore_barrier()

    def start_idx(ch, d):
        chg = c * DCHUNKS + ch
        pltpu.async_copy(col_hbm.at[s, chg], col_v.at[d], isem)
        pltpu.async_copy(ew_hbm.at[s, pl.ds(chg * CE, CE)],
                         ew_v.at[pl.ds(d * CE, CE)], isem)

    def wait_idx(d):
        pltpu.make_async_copy(col_hbm.at[s, 0], col_v.at[d], isem).wait()
        pltpu.make_async_copy(ew_hbm.at[s, pl.ds(0, CE)],
                              ew_v.at[pl.ds(d * CE, CE)], isem).wait()

    def start_scatter(r, d):
        for j in range(CG):
            pltpu.async_copy(rows_v.at[pl.ds(r * CE + j * GRP, GRP)],
                             z_sh.at[col_v.at[d, j]], ssem, add=True)

    def wait_scatter(r):
        pltpu.make_async_copy(rows_v.at[pl.ds(r * CE, CE)],
                              z_sh.at[pl.ds(0, CE)], ssem).wait()

    start_idx(0, 0)
    start_idx(1, 1)

    @pl.loop(0, DCHUNKS)
    def _chunk(i):
        r = lax.bitwise_and(i, 1)
        d = lax.bitwise_and(i, 3)

        @pl.when(i >= 2)
        def _():
            wait_scatter(lax.bitwise_and(i, 1))

        wait_idx(d)

        rbase = r * CE
        ebase = d * CE

        @pl.loop(0, CE // 16)
        def _fill(g):
            ew16 = ew_v[pl.ds(ebase + g * 16, 16)]
            for k in range(16):
                e = rbase + g * 16 + k
                w16 = ew16.at[jnp.full((16,), k, jnp.int32)].get(
                    mode="promise_in_bounds")
                v0 = rows_v[e, pl.ds(0, HF)]
                rows_v[e, pl.ds(0, HF)] = v0 * 0.0 + w16

        @pl.when(i + 2 < DCHUNKS)
        def _():
            start_idx(i + 2, lax.bitwise_and(i + 2, 3))

        start_scatter(r, d)

    wait_scatter(lax.bitwise_and(DCHUNKS - 2, 1))
    wait_scatter(lax.bitwise_and(DCHUNKS - 1, 1))

    plsc.subcore_barrier()
    pltpu.sync_copy(z_sh.at[pl.ds(base_row, ROWS_PER_TILE)],
                    zp_hbm.at[c, pl.ds(base_row, ROWS_PER_TILE)])


def _deg_call(col_r2, ew_r2, zeros_rt):
    k = pl.kernel(
        _deg_body,
        out_type=jax.ShapeDtypeStruct((NC, NP, HF), jnp.float32),
        mesh=_sc_mesh(),
        compiler_params=pltpu.CompilerParams(use_tc_tiling_on_sc=False),
        scratch_types=[
            pltpu.VMEM((4, CG, GRP), jnp.int32),
            pltpu.VMEM((4 * CE,), jnp.float32),
            pltpu.VMEM((2 * CE, HF), jnp.float32),
            pltpu.VMEM_SHARED((NP, HF), jnp.float32),
            pltpu.SemaphoreType.DMA,
            pltpu.SemaphoreType.DMA,
        ],
    )
    args = [pltpu.with_memory_space_constraint(a, pltpu.HBM)
            for a in (col_r2, ew_r2, zeros_rt)]
    return k(*args)


BLK = 2048
GRID = NP // BLK


def _mm(a, b):
    return lax.dot_general(a, b, (((1,), (0,)), ((), ())),
                           preferred_element_type=jnp.float32)


def _tc_ab_body(c_ref, x_ref, degp_ref, we1_ref, be1_ref, we2_ref, be2_ref,
                w1a_ref, w1b_ref, y1s_ref, dinv_ref, xw1_ref):
    emb = _mm(jax.nn.relu(_mm(c_ref[...], we1_ref[...]) + be1_ref[...]),
              we2_ref[...]) + be2_ref[...]
    xw1 = _mm(x_ref[...], w1a_ref[...]) + _mm(emb, w1b_ref[...])
    xw1_ref[...] = xw1
    deg = degp_ref[0, :, 0:1] + degp_ref[1, :, 0:1] + 1.0
    dinv = lax.rsqrt(deg)
    dinv_ref[...] = dinv
    y1 = xw1 * dinv
    y1s_ref[0] = y1[:, :HF]
    y1s_ref[1] = y1[:, HF:]


def _tc_c_body(zp_ref, dinv_ref, xw1_ref, b1_ref, w2_ref, y2s_ref, xw2_ref):
    dinv = dinv_ref[...]
    z1 = jnp.concatenate([zp_ref[0], zp_ref[1]], axis=1)
    h1 = jax.nn.relu((z1 + dinv * xw1_ref[...]) * dinv + b1_ref[...])
    xw2 = _mm(h1, w2_ref[...])
    xw2_ref[...] = xw2
    y2 = xw2 * dinv
    y2s_ref[0] = y2[:, :HF]
    y2s_ref[1] = y2[:, HF:]


def _tc_d_body(zp_ref, dinv_ref, xw2_ref, b2_ref, wf_ref, bf_ref, out_ref):
    dinv = dinv_ref[...]
    z2 = jnp.concatenate([zp_ref[0], zp_ref[1]], axis=1)
    h2 = jax.nn.relu((z2 + dinv * xw2_ref[...]) * dinv + b2_ref[...])
    out_ref[...] = _mm(h2, wf_ref[...]) + bf_ref[...]


def _row_spec(width):
    return pl.BlockSpec((BLK, width), lambda i: (i, 0))


def _part_spec(width):
    return pl.BlockSpec((NC, BLK, width), lambda i: (0, i, 0))


def _full_spec(shape):
    nd = len(shape)
    return pl.BlockSpec(shape, lambda i: (0,) * nd)


def kernel(x, c, ei, ew, W_enc1, b_enc1, W_enc2, b_enc2, W1, b1, W2, b2, Wf, bf):
    f32 = jnp.float32
    x = x.astype(f32)
    c = c.astype(f32)
    ew = ew.astype(f32)

    pad_e = EP - E
    row = jnp.concatenate([ei[0].astype(jnp.int32),
                           jnp.zeros((pad_e,), jnp.int32)])
    col = jnp.concatenate([ei[1].astype(jnp.int32),
                           jnp.zeros((pad_e,), jnp.int32)])
    ewp = jnp.concatenate([ew, jnp.zeros((pad_e,), f32)])
    row_t = row.reshape(NS, CHUNKS, CG, GRP)
    row_r2 = jnp.stack([row_t, row_t + NP])
    col_r2 = col.reshape(NS, CHUNKS, CG, GRP)
    ew_r2 = ewp.reshape(NS, CHUNKS * CE)

    pad_n = NP - N
    xp = jnp.concatenate([x, jnp.zeros((pad_n, x.shape[1]), f32)])
    cp = jnp.concatenate([c, jnp.zeros((pad_n, c.shape[1]), f32)])

    zeros_rt = jnp.zeros((ROWS_PER_TILE, HF), f32)

    be1 = b_enc1.reshape(1, -1)
    be2 = b_enc2.reshape(1, -1)
    W1a = W1[:3]
    W1b = W1[3:]
    b1r = b1.reshape(1, -1)
    b2r = b2.reshape(1, -1)
    bfr = bf.reshape(1, -1)

    degp = _deg_call(col_r2, ew_r2, zeros_rt)

    y1s, dinv, xw1 = pl.pallas_call(
        _tc_ab_body,
        grid=(GRID,),
        in_specs=[_row_spec(2), _row_spec(3), _part_spec(HF),
                  _full_spec(W_enc1.shape), _full_spec(be1.shape),
                  _full_spec(W_enc2.shape), _full_spec(be2.shape),
                  _full_spec(W1a.shape), _full_spec(W1b.shape)],
        out_specs=[_part_spec(HF), _row_spec(1), _row_spec(32)],
        out_shape=[jax.ShapeDtypeStruct((NC, NP, HF), f32),
                   jax.ShapeDtypeStruct((NP, 1), f32),
                   jax.ShapeDtypeStruct((NP, 32), f32)],
    )(cp, xp, degp, W_enc1, be1, W_enc2, be2, W1a, W1b)

    z1p = _msg_call(y1s.reshape(2 * NP, HF), row_r2, col_r2, ew_r2, zeros_rt)

    y2s, xw2 = pl.pallas_call(
        _tc_c_body,
        grid=(GRID,),
        in_specs=[_part_spec(HF), _row_spec(1), _row_spec(32),
                  _full_spec(b1r.shape), _full_spec(W2.shape)],
        out_specs=[_part_spec(HF), _row_spec(32)],
        out_shape=[jax.ShapeDtypeStruct((NC, NP, HF), f32),
                   jax.ShapeDtypeStruct((NP, 32), f32)],
    )(z1p, dinv, xw1, b1r, W2)

    z2p = _msg_call(y2s.reshape(2 * NP, HF), row_r2, col_r2, ew_r2, zeros_rt)

    out = pl.pallas_call(
        _tc_d_body,
        grid=(GRID,),
        in_specs=[_part_spec(HF), _row_spec(1), _row_spec(32),
                  _full_spec(b2r.shape), _full_spec(Wf.shape),
                  _full_spec(bfr.shape)],
        out_specs=_row_spec(1),
        out_shape=jax.ShapeDtypeStruct((NP, 1), f32),
    )(z2p, dinv, xw2, b2r, Wf, bfr)

    return out[:N]
